# baseline (speedup 1.0000x reference)
import jax
import jax.numpy as jnp
from jax import lax
from jax.experimental import pallas as pl
from jax.experimental.pallas import tpu as pltpu
from jax.experimental.pallas import tpu_sc as plsc

DEPTH = 1000
ROWS = 4096 * 26
NUM_CORES = 2
NUM_SUBCORES = 16
NW = NUM_CORES * NUM_SUBCORES
ROWS_PER_W = ROWS // NW
BLK_ROWS = 32
NBUF = 2
NBLK = ROWS_PER_W // BLK_ROWS
BLK_WORDS = BLK_ROWS * DEPTH
LANE = 16


def _onehot_sc_body(idx_hbm, onoff_hbm, out_hbm,
                    idx_v, onoff_v, fill_v, pos_v, val_v, shared,
                    sem0, sem1):
    cid = lax.axis_index("c")
    sid = lax.axis_index("s")
    wid = sid * NUM_CORES + cid
    row0 = wid * ROWS_PER_W
    out_base = row0 * DEPTH
    sh_base = sid * (NBUF * BLK_WORDS)
    sems = (sem0, sem1)

    pltpu.sync_copy(idx_hbm.at[pl.ds(row0, ROWS_PER_W)], idx_v)
    pltpu.sync_copy(onoff_hbm, onoff_v)
    on_vec = onoff_v[pl.ds(0, LANE)]
    off_vec = onoff_v[pl.ds(LANE, LANE)]
    lane = lax.iota(jnp.int32, LANE)

    def fill_body(i, c):
        fill_v[pl.ds(i * LANE, LANE)] = off_vec
        return c
    lax.fori_loop(0, BLK_WORDS // LANE, fill_body, 0)
    for b in range(NBUF):
        pltpu.sync_copy(fill_v, shared.at[pl.ds(sh_base + b * BLK_WORDS,
                                                BLK_WORDS)])

    def prep_positions(g, b):
        for j in range(BLK_ROWS // LANE):
            idx16 = idx_v[pl.ds(g * BLK_ROWS + j * LANE, LANE)]
            pos_v[pl.ds(j * LANE, LANE)] = (
                (lane + j * LANE) * DEPTH + idx16
                + (sh_base + b * BLK_WORDS))

    def set_vals(vec):
        for j in range(BLK_ROWS // LANE):
            val_v[pl.ds(j * LANE, LANE)] = vec

    def scatter_vals():
        pltpu.sync_copy(val_v, shared.at[pos_v])

    def start_dma(b, g):
        pltpu.async_copy(
            shared.at[pl.ds(sh_base + b * BLK_WORDS, BLK_WORDS)],
            out_hbm.at[pl.ds(out_base + g * BLK_WORDS, BLK_WORDS)],
            sems[b])

    def wait_dma(b, g):
        pltpu.make_async_copy(
            shared.at[pl.ds(sh_base + b * BLK_WORDS, BLK_WORDS)],
            out_hbm.at[pl.ds(out_base + g * BLK_WORDS, BLK_WORDS)],
            sems[b]).wait()

    for b in range(NBUF):
        prep_positions(b, b)
        set_vals(on_vec)
        scatter_vals()
        start_dma(b, b)

    def body(i, c):
        for b in range(NBUF):
            g = NBUF * i + b
            wait_dma(b, g - NBUF)
            prep_positions(g - NBUF, b)
            set_vals(off_vec)
            scatter_vals()
            prep_positions(g, b)
            set_vals(on_vec)
            scatter_vals()
            start_dma(b, g)
        return c

    lax.fori_loop(1, NBLK // NBUF, body, 0)

    for b in range(NBUF):
        wait_dma(b, NBLK - NBUF + b)


def kernel(indices, on_value, off_value):
    idx = indices.reshape(-1).astype(jnp.int32)
    onoff = jnp.concatenate([
        jnp.full((LANE,), on_value, jnp.float32),
        jnp.full((LANE,), off_value, jnp.float32),
    ])
    mesh = plsc.VectorSubcoreMesh(
        core_axis_name="c", subcore_axis_name="s",
        num_cores=NUM_CORES, num_subcores=NUM_SUBCORES)
    out = pl.kernel(
        _onehot_sc_body,
        out_type=jax.ShapeDtypeStruct((ROWS * DEPTH,), jnp.float32),
        mesh=mesh,
        compiler_params=pltpu.CompilerParams(needs_layout_passes=False),
        scratch_types=(
            [pltpu.VMEM((ROWS_PER_W,), jnp.int32),
             pltpu.VMEM((2 * LANE,), jnp.float32),
             pltpu.VMEM((BLK_WORDS,), jnp.float32),
             pltpu.VMEM((BLK_ROWS,), jnp.int32),
             pltpu.VMEM((BLK_ROWS,), jnp.float32),
             pltpu.VMEM_SHARED((NUM_SUBCORES * NBUF * BLK_WORDS,),
                               jnp.float32)]
            + [pltpu.SemaphoreType.DMA] * NBUF
        ),
    )(idx, onoff)
    return out.reshape(indices.shape + (DEPTH,))


# speedup vs baseline: 9.7959x; 9.7959x over previous
import jax
import jax.numpy as jnp
from jax import lax
from jax.experimental import pallas as pl
from jax.experimental.pallas import tpu as pltpu

DEPTH = 1000
N = 4096
J = 26
D_BLK = 200


def _onehot_tc_body(on_ref, off_ref, idx_ref, out_ref):
    d0 = pl.program_id(1) * D_BLK
    iota_d = lax.broadcasted_iota(jnp.int32, (D_BLK, N), 0) + d0
    mask = iota_d == idx_ref[0, 0][None, :]
    out_ref[0] = jnp.where(mask, on_ref[0, 0], off_ref[0, 0])


def kernel(indices, on_value, off_value):
    idx_t = indices.T.astype(jnp.int32).reshape(J, 1, N)
    on2 = on_value.reshape(1, 1)
    off2 = off_value.reshape(1, 1)
    out = pl.pallas_call(
        _onehot_tc_body,
        grid=(J, DEPTH // D_BLK),
        in_specs=[
            pl.BlockSpec(memory_space=pltpu.SMEM),
            pl.BlockSpec(memory_space=pltpu.SMEM),
            pl.BlockSpec((1, 1, N), lambda j, t: (j, 0, 0)),
        ],
        out_specs=pl.BlockSpec((1, D_BLK, N), lambda j, t: (j, t, 0)),
        out_shape=jax.ShapeDtypeStruct((J, DEPTH, N), jnp.float32),
    )(on2, off2, idx_t)
    return jnp.transpose(out, (2, 0, 1))
